# baseline (speedup 1.0000x reference)
import functools

import jax
import jax.numpy as jnp
from jax import lax
from jax.experimental import pallas as pl
from jax.experimental.pallas import tpu as pltpu
from jax.experimental.pallas import tpu_sc as plsc

N_EXP = 16
TOPK = 2
ALPHA = 0.01
HID = 2048
ROWS = 8192
BLK = 1024

L = 16
N_SUB = 16
TPT = ROWS // N_SUB
N_GRP = TPT // L
LOSS_SCALE = ALPHA * N_EXP / (float(ROWS) * float(ROWS * TOPK))


def _lane_allsum(v):
    for sh in (8, 4, 2, 1):
        idx = lax.iota(jnp.int32, L) ^ sh
        v = v + lax.gather(
            v, idx[:, None],
            dimension_numbers=lax.GatherDimensionNumbers(
                offset_dims=(), collapsed_slice_dims=(0,),
                start_index_map=(0,)),
            slice_sizes=(1,), unique_indices=True,
            mode=lax.GatherScatterMode.PROMISE_IN_BOUNDS)
    return v


def _logits_body(x_ref, w_ref, lt_ref):
    lt_ref[...] = lax.dot_general(w_ref[...], x_ref[...],
                                  (((1,), (1,)), ((), ())),
                                  preferred_element_type=jnp.float32)


def _tc_logits(xf, w):
    return pl.pallas_call(
        _logits_body,
        grid=(ROWS // BLK,),
        in_specs=[
            pl.BlockSpec((BLK, HID), lambda i: (i, 0)),
            pl.BlockSpec((N_EXP, HID), lambda i: (0, 0)),
        ],
        out_specs=pl.BlockSpec((N_EXP, BLK), lambda i: (0, i)),
        out_shape=jax.ShapeDtypeStruct((N_EXP, ROWS), jnp.float32),
        compiler_params=pltpu.CompilerParams(
            dimension_semantics=("arbitrary",),
        ),
    )(xf, w)


@functools.partial(
    pl.kernel,
    out_type=[
        jax.ShapeDtypeStruct((TOPK, ROWS), jnp.int32),
        jax.ShapeDtypeStruct((TOPK, ROWS), jnp.float32),
        jax.ShapeDtypeStruct((N_SUB, 2, N_EXP), jnp.float32),
    ],
    mesh=plsc.VectorSubcoreMesh(core_axis_name="c", subcore_axis_name="s",
                                num_cores=1),
    scratch_types=[
        pltpu.VMEM((N_EXP, TPT), jnp.float32),
        pltpu.VMEM((TPT,), jnp.int32),
        pltpu.VMEM((TPT,), jnp.int32),
        pltpu.VMEM((TPT,), jnp.float32),
        pltpu.VMEM((TPT,), jnp.float32),
        pltpu.VMEM((N_EXP, L), jnp.float32),
        pltpu.VMEM((N_EXP, L), jnp.float32),
        pltpu.VMEM((2, N_EXP), jnp.float32),
    ],
)
def _sc_gate(lt_hbm, idx_hbm, wgt_hbm, parts_hbm,
             slab, i1_v, i2_v, w1_v, w2_v, pi_v, cnt_v, part_v):
    wid = lax.axis_index("s")
    base = wid * TPT
    pltpu.sync_copy(lt_hbm.at[:, pl.ds(base, TPT)], slab)

    zero = jnp.zeros((L,), jnp.float32)
    for e in range(N_EXP):
        pi_v[e, :] = zero
        cnt_v[e, :] = zero

    def grp(g, carry):
        b = g * L
        vs = [slab[e, pl.ds(b, L)] for e in range(N_EXP)]
        m1 = vs[0]
        i1 = jnp.zeros((L,), jnp.int32)
        m2 = jnp.full((L,), -jnp.inf, jnp.float32)
        i2 = jnp.zeros((L,), jnp.int32)
        for e in range(1, N_EXP):
            v = vs[e]
            es = jnp.full((L,), e, jnp.int32)
            gt1 = v > m1
            gt2 = v > m2
            m2 = jnp.where(gt1, m1, jnp.where(gt2, v, m2))
            i2 = jnp.where(gt1, i1, jnp.where(gt2, es, i2))
            m1 = jnp.where(gt1, v, m1)
            i1 = jnp.where(gt1, es, i1)

        den = jnp.zeros((L,), jnp.float32)
        ts = []
        for e in range(N_EXP):
            t = jnp.exp(vs[e] - m1)
            ts.append(t)
            den = den + t
        rden = 1.0 / den
        one = jnp.full((L,), 1.0, jnp.float32)
        zer = jnp.zeros((L,), jnp.float32)
        for e in range(N_EXP):
            pi_v[e, :] = pi_v[e, :] + ts[e] * rden
            es = jnp.full((L,), e, jnp.int32)
            inc = (jnp.where(i1 == es, one, zer)
                   + jnp.where(i2 == es, one, zer))
            cnt_v[e, :] = cnt_v[e, :] + inc

        e2 = jnp.exp(m2 - m1)
        w1 = 1.0 / (1.0 + e2)
        w2 = 1.0 - w1
        i1_v[pl.ds(b, L)] = i1
        i2_v[pl.ds(b, L)] = i2
        w1_v[pl.ds(b, L)] = w1
        w2_v[pl.ds(b, L)] = w2
        return carry

    lax.fori_loop(0, N_GRP, grp, 0)

    pltpu.sync_copy(i1_v, idx_hbm.at[0, pl.ds(base, TPT)])
    pltpu.sync_copy(i2_v, idx_hbm.at[1, pl.ds(base, TPT)])
    pltpu.sync_copy(w1_v, wgt_hbm.at[0, pl.ds(base, TPT)])
    pltpu.sync_copy(w2_v, wgt_hbm.at[1, pl.ds(base, TPT)])

    iota = lax.iota(jnp.int32, L)
    one = jnp.full((L,), 1.0, jnp.float32)
    zer = jnp.zeros((L,), jnp.float32)
    pi_tot = jnp.zeros((L,), jnp.float32)
    cnt_tot = jnp.zeros((L,), jnp.float32)
    for e in range(N_EXP):
        oh = jnp.where(iota == e, one, zer)
        pi_tot = pi_tot + oh * _lane_allsum(pi_v[e, :])
        cnt_tot = cnt_tot + oh * _lane_allsum(cnt_v[e, :])
    part_v[0, :] = pi_tot
    part_v[1, :] = cnt_tot
    pltpu.sync_copy(part_v, parts_hbm.at[wid])


def _loss_body(parts_ref, loss_ref):
    p = parts_ref[...]
    pi_tot = jnp.sum(p[:, 0, :], axis=0)
    cnt_tot = jnp.sum(p[:, 1, :], axis=0)
    loss_ref[...] = (jnp.sum(pi_tot * cnt_tot) * LOSS_SCALE).reshape(1, 1)


def _tc_loss(parts):
    return pl.pallas_call(
        _loss_body,
        out_shape=jax.ShapeDtypeStruct((1, 1), jnp.float32),
    )(parts)


@jax.jit
def kernel(x, weight):
    xf = x.reshape(-1, HID)
    lt = _tc_logits(xf, weight)
    idx_t, wgt_t, parts = _sc_gate(lt)
    loss = _tc_loss(parts)
    return idx_t.T, wgt_t.T, loss[0, 0]


# speedup vs baseline: 1.5586x; 1.5586x over previous
import functools

import jax
import jax.numpy as jnp
from jax import lax
from jax.experimental import pallas as pl
from jax.experimental.pallas import tpu as pltpu

N_EXP = 16
TOPK = 2
ALPHA = 0.01
HID = 2048
ROWS = 8192
BLK = 1024
KSPLIT = 2
KB = HID // KSPLIT


def _gate_body(x_ref, w_ref, idx_ref, wgt_ref, loss_ref,
               lt_acc, pi_acc, cnt_acc):
    i = pl.program_id(0)
    j = pl.program_id(1)
    nsteps = pl.num_programs(0)

    part = lax.dot_general(w_ref[...], x_ref[...], (((1,), (1,)), ((), ())),
                           preferred_element_type=jnp.float32)

    @pl.when(j == 0)
    def _first():
        lt_acc[...] = part

    @pl.when(j == KSPLIT - 1)
    def _gate():
        lt = lt_acc[...] + part

        m1 = jnp.max(lt, axis=0, keepdims=True)
        e = jnp.exp(lt - m1)
        denom = jnp.sum(e, axis=0, keepdims=True)
        rdenom = 1.0 / denom

        iota = lax.broadcasted_iota(jnp.int32, (N_EXP, BLK), 0)
        is1 = lt == m1
        idx1 = jnp.min(jnp.where(is1, iota, N_EXP), axis=0, keepdims=True)
        mask1 = iota == idx1
        neg = jnp.float32(-jnp.inf)
        l2 = jnp.where(mask1, neg, lt)
        m2 = jnp.max(l2, axis=0, keepdims=True)
        idx2 = jnp.min(jnp.where(l2 == m2, iota, N_EXP), axis=0, keepdims=True)
        mask2 = iota == idx2

        s1 = rdenom
        s2 = jnp.exp(m2 - m1) * rdenom
        d12 = s1 + s2 + 1e-20
        w1 = s1 / d12
        w2 = s2 / d12

        idx_ref[...] = jnp.concatenate([idx1, idx2], axis=0)
        wgt_ref[...] = jnp.concatenate([w1, w2], axis=0)

        pi_part = jnp.sum(e * rdenom, axis=1, keepdims=True)
        cnt_part = jnp.sum(mask1.astype(jnp.float32)
                           + mask2.astype(jnp.float32),
                           axis=1, keepdims=True)

        @pl.when(i == 0)
        def _init():
            pi_acc[...] = jnp.zeros_like(pi_acc)
            cnt_acc[...] = jnp.zeros_like(cnt_acc)

        pi_acc[...] += pi_part
        cnt_acc[...] += cnt_part

        @pl.when(i == nsteps - 1)
        def _fin():
            scale = ALPHA * N_EXP / (float(ROWS) * float(ROWS * TOPK))
            loss_ref[...] = jnp.sum(pi_acc[...] * cnt_acc[...],
                                    keepdims=True).reshape(1, 1) * scale


@jax.jit
def kernel(x, weight):
    xf = x.reshape(-1, HID)
    grid = (ROWS // BLK, KSPLIT)
    idx_t, wgt_t, loss = pl.pallas_call(
        _gate_body,
        grid=grid,
        in_specs=[
            pl.BlockSpec((BLK, KB), lambda i, j: (i, j)),
            pl.BlockSpec((N_EXP, KB), lambda i, j: (0, j)),
        ],
        out_specs=[
            pl.BlockSpec((TOPK, BLK), lambda i, j: (0, i)),
            pl.BlockSpec((TOPK, BLK), lambda i, j: (0, i)),
            pl.BlockSpec((1, 1), lambda i, j: (0, 0)),
        ],
        out_shape=[
            jax.ShapeDtypeStruct((TOPK, ROWS), jnp.int32),
            jax.ShapeDtypeStruct((TOPK, ROWS), jnp.float32),
            jax.ShapeDtypeStruct((1, 1), jnp.float32),
        ],
        scratch_shapes=[
            pltpu.VMEM((N_EXP, BLK), jnp.float32),
            pltpu.VMEM((N_EXP, 1), jnp.float32),
            pltpu.VMEM((N_EXP, 1), jnp.float32),
        ],
        compiler_params=pltpu.CompilerParams(
            dimension_semantics=("arbitrary", "arbitrary"),
        ),
    )(xf, weight)
    return idx_t.T, wgt_t.T, loss.reshape(())


# speedup vs baseline: 1.7752x; 1.1390x over previous
import jax
import jax.numpy as jnp
from jax import lax
from jax.experimental import pallas as pl
from jax.experimental.pallas import tpu as pltpu

N_EXP = 16
TOPK = 2
ALPHA = 0.01
HID = 2048
ROWS = 8192
CH = 512
NBUF = 4
NCHUNK = ROWS // CH


def _gate_body(x_hbm, w_ref, idx_ref, wgt_ref, loss_ref, bufs, sems,
               pi_acc, cnt_acc):
    w = w_ref[...]

    def start(c):
        slot = c % NBUF
        pltpu.make_async_copy(
            x_hbm.at[pl.ds(c * CH, CH), :], bufs.at[slot], sems.at[slot]
        ).start()

    def wait(c):
        slot = c % NBUF
        pltpu.make_async_copy(
            x_hbm.at[pl.ds(c * CH, CH), :], bufs.at[slot], sems.at[slot]
        ).wait()

    for c in range(NBUF):
        start(c)

    pi = jnp.zeros((N_EXP, 1), jnp.float32)
    cnt = jnp.zeros((N_EXP, 1), jnp.float32)

    for c in range(NCHUNK):
        wait(c)
        x = bufs[c % NBUF]
        lt = lax.dot_general(w, x, (((1,), (1,)), ((), ())),
                             preferred_element_type=jnp.float32)
        if c + NBUF < NCHUNK:
            start(c + NBUF)

        m1 = jnp.max(lt, axis=0, keepdims=True)
        e = jnp.exp(lt - m1)
        denom = jnp.sum(e, axis=0, keepdims=True)
        rdenom = 1.0 / denom

        iota = lax.broadcasted_iota(jnp.int32, (N_EXP, CH), 0)
        is1 = lt == m1
        idx1 = jnp.min(jnp.where(is1, iota, N_EXP), axis=0, keepdims=True)
        mask1 = iota == idx1
        neg = jnp.float32(-jnp.inf)
        l2 = jnp.where(mask1, neg, lt)
        m2 = jnp.max(l2, axis=0, keepdims=True)
        idx2 = jnp.min(jnp.where(l2 == m2, iota, N_EXP), axis=0, keepdims=True)
        mask2 = iota == idx2

        s1 = rdenom
        s2 = jnp.exp(m2 - m1) * rdenom
        d12 = s1 + s2 + 1e-20
        w1 = s1 / d12
        w2 = s2 / d12

        idx_ref[:, pl.ds(c * CH, CH)] = jnp.concatenate([idx1, idx2], axis=0)
        wgt_ref[:, pl.ds(c * CH, CH)] = jnp.concatenate([w1, w2], axis=0)

        pi = pi + jnp.sum(e * rdenom, axis=1, keepdims=True)
        cnt = cnt + jnp.sum(mask1.astype(jnp.float32)
                            + mask2.astype(jnp.float32),
                            axis=1, keepdims=True)

    del pi_acc, cnt_acc
    scale = ALPHA * N_EXP / (float(ROWS) * float(ROWS * TOPK))
    loss_ref[...] = jnp.sum(pi * cnt, keepdims=True).reshape(1, 1) * scale


@jax.jit
def kernel(x, weight):
    xf = x.reshape(-1, HID)
    idx_t, wgt_t, loss = pl.pallas_call(
        _gate_body,
        in_specs=[
            pl.BlockSpec(memory_space=pl.ANY),
            pl.BlockSpec(memory_space=pltpu.VMEM),
        ],
        out_specs=[
            pl.BlockSpec(memory_space=pltpu.VMEM),
            pl.BlockSpec(memory_space=pltpu.VMEM),
            pl.BlockSpec(memory_space=pltpu.VMEM),
        ],
        out_shape=[
            jax.ShapeDtypeStruct((TOPK, ROWS), jnp.int32),
            jax.ShapeDtypeStruct((TOPK, ROWS), jnp.float32),
            jax.ShapeDtypeStruct((1, 1), jnp.float32),
        ],
        scratch_shapes=[
            pltpu.VMEM((NBUF, CH, HID), jnp.float32),
            pltpu.SemaphoreType.DMA((NBUF,)),
            pltpu.VMEM((N_EXP, 1), jnp.float32),
            pltpu.VMEM((N_EXP, 1), jnp.float32),
        ],
    )(xf, weight)
    return idx_t.T, wgt_t.T, loss.reshape(())
